# baseline (speedup 1.0000x reference)
import functools

import jax
import jax.numpy as jnp
from jax import lax
from jax.experimental import pallas as pl
from jax.experimental.pallas import tpu as pltpu
from jax.experimental.pallas import tpu_sc as plsc

_NUM_CLASSES = 100000
_FEAT_DIM = 128
_BATCH = 16384
_LAMBDA_C = 0.001

_NC = 2
_NS = 16
_NW = _NC * _NS
_PER_W = _BATCH // _NW
_CHUNK = 64
_NCHUNK = _PER_W // _CHUNK
_NBUF = 2
_L = 16
_NJ = _FEAT_DIM // _L


def _sc_body(feats_hbm, labels_hbm, centers_hbm, out_hbm,
             idx_v, acc_v, *bufs_and_sems):
    feats = bufs_and_sems[0:_NBUF]
    rows = bufs_and_sems[_NBUF:2 * _NBUF]
    gsems = bufs_and_sems[2 * _NBUF:3 * _NBUF]
    fsems = bufs_and_sems[3 * _NBUF:4 * _NBUF]

    wid = lax.axis_index("s") * _NC + lax.axis_index("c")
    base = wid * _PER_W

    def start(c, b):
        off = pl.multiple_of(c * _CHUNK, 8)
        pltpu.async_copy(
            centers_hbm.at[idx_v.at[pl.ds(off, _CHUNK)]], rows[b], gsems[b])
        pltpu.async_copy(
            feats_hbm.at[pl.ds(base + c * _CHUNK, _CHUNK)], feats[b], fsems[b])

    def wait(b):
        pltpu.make_async_copy(
            feats_hbm.at[pl.ds(0, _CHUNK)], rows[b], gsems[b]).wait()
        pltpu.make_async_copy(
            feats_hbm.at[pl.ds(0, _CHUNK)], feats[b], fsems[b]).wait()

    pltpu.sync_copy(labels_hbm.at[pl.ds(base, _PER_W)], idx_v)
    for b in range(_NBUF):
        start(b, b)

    accs0 = tuple(jnp.zeros((_L,), jnp.float32) for _ in range(_NJ))

    @pl.loop(0, _NCHUNK, step=_NBUF, init_carry=accs0)
    def accs(g, accs):
        for b in range(_NBUF):
            wait(b)
            f_v, r_v = feats[b], rows[b]

            @plsc.parallel_loop(0, _CHUNK, carry=accs)
            def accs(i, a):
                out = []
                for j in range(_NJ):
                    d = (f_v[i, pl.ds(j * _L, _L)]
                         - r_v[i, pl.ds(j * _L, _L)])
                    out.append(a[j] + d * d)
                return tuple(out)

            c2 = g + b + _NBUF

            @pl.when(c2 < _NCHUNK)
            def _():
                start(c2, b)
        return accs

    total = accs[0]
    for j in range(1, _NJ):
        total = total + accs[j]
    acc_v[...] = total * (_LAMBDA_C / float(_BATCH * _FEAT_DIM))
    pltpu.sync_copy(acc_v, out_hbm.at[wid])


@jax.jit
def _center_loss_sc(features, labels_i32, centers):
    mesh = plsc.VectorSubcoreMesh(core_axis_name="c", subcore_axis_name="s")
    partials = pl.kernel(
        _sc_body,
        out_type=jax.ShapeDtypeStruct((_NW, _L), jnp.float32),
        mesh=mesh,
        scratch_types=(
            [pltpu.VMEM((_PER_W,), jnp.int32),
             pltpu.VMEM((_L,), jnp.float32)]
            + [pltpu.VMEM((_CHUNK, _FEAT_DIM), jnp.float32)
               for _ in range(2 * _NBUF)]
            + [pltpu.SemaphoreType.DMA for _ in range(2 * _NBUF)]
        ),
    )(features, labels_i32, centers)
    return jnp.sum(partials)


def kernel(features, labels, centers):
    return _center_loss_sc(features, labels.astype(jnp.int32), centers)


# speedup vs baseline: 1.0606x; 1.0606x over previous
import functools

import jax
import jax.numpy as jnp
from jax import lax
from jax.experimental import pallas as pl
from jax.experimental.pallas import tpu as pltpu
from jax.experimental.pallas import tpu_sc as plsc

_NUM_CLASSES = 100000
_FEAT_DIM = 128
_BATCH = 16384
_LAMBDA_C = 0.001

_NC = 2
_NS = 16
_NW = _NC * _NS
_PER_W = _BATCH // _NW
_CHUNK = 32
_NCHUNK = _PER_W // _CHUNK
_NBUF = 4
_L = 16
_NJ = _FEAT_DIM // _L


def _sc_body(feats_hbm, labels_hbm, centers_hbm, out_hbm,
             idx_v, acc_v, *bufs_and_sems):
    feats = bufs_and_sems[0:_NBUF]
    rows = bufs_and_sems[_NBUF:2 * _NBUF]
    gsems = bufs_and_sems[2 * _NBUF:3 * _NBUF]
    fsems = bufs_and_sems[3 * _NBUF:4 * _NBUF]

    wid = lax.axis_index("s") * _NC + lax.axis_index("c")
    base = wid * _PER_W

    def start(c, b):
        off = pl.multiple_of(c * _CHUNK, 8)
        pltpu.async_copy(
            centers_hbm.at[idx_v.at[pl.ds(off, _CHUNK)]], rows[b], gsems[b])
        pltpu.async_copy(
            feats_hbm.at[pl.ds(base + c * _CHUNK, _CHUNK)], feats[b], fsems[b])

    def wait(b):
        pltpu.make_async_copy(
            feats_hbm.at[pl.ds(0, _CHUNK)], rows[b], gsems[b]).wait()
        pltpu.make_async_copy(
            feats_hbm.at[pl.ds(0, _CHUNK)], feats[b], fsems[b]).wait()

    pltpu.sync_copy(labels_hbm.at[pl.ds(base, _PER_W)], idx_v)
    for b in range(_NBUF):
        start(b, b)

    accs0 = tuple(jnp.zeros((_L,), jnp.float32) for _ in range(_NJ))

    @pl.loop(0, _NCHUNK, step=_NBUF, init_carry=accs0)
    def accs(g, accs):
        for b in range(_NBUF):
            wait(b)
            f_v, r_v = feats[b], rows[b]

            @plsc.parallel_loop(0, _CHUNK, unroll=2, carry=accs)
            def accs(i, a):
                out = []
                for j in range(_NJ):
                    d = (f_v[i, pl.ds(j * _L, _L)]
                         - r_v[i, pl.ds(j * _L, _L)])
                    out.append(a[j] + d * d)
                return tuple(out)

            c2 = g + b + _NBUF

            @pl.when(c2 < _NCHUNK)
            def _():
                start(c2, b)
        return accs

    total = accs[0]
    for j in range(1, _NJ):
        total = total + accs[j]
    acc_v[...] = total * (_LAMBDA_C / float(_BATCH * _FEAT_DIM))
    pltpu.sync_copy(acc_v, out_hbm.at[wid])


@jax.jit
def _center_loss_sc(features, labels_i32, centers):
    mesh = plsc.VectorSubcoreMesh(core_axis_name="c", subcore_axis_name="s")
    partials = pl.kernel(
        _sc_body,
        out_type=jax.ShapeDtypeStruct((_NW, _L), jnp.float32),
        mesh=mesh,
        scratch_types=(
            [pltpu.VMEM((_PER_W,), jnp.int32),
             pltpu.VMEM((_L,), jnp.float32)]
            + [pltpu.VMEM((_CHUNK, _FEAT_DIM), jnp.float32)
               for _ in range(2 * _NBUF)]
            + [pltpu.SemaphoreType.DMA for _ in range(2 * _NBUF)]
        ),
    )(features, labels_i32, centers)
    return jnp.sum(partials)


def kernel(features, labels, centers):
    return _center_loss_sc(features, labels.astype(jnp.int32), centers)
